# Initial kernel scaffold; baseline (speedup 1.0000x reference)
import jax
import jax.numpy as jnp
from jax.experimental import pallas as pl


def kernel(description_idx, values, embedded_descriptions, W_val, b_val, W_proj, b_proj):
    raise NotImplementedError("write your pallas kernel here")


# speedup vs baseline: 4.7541x; 4.7541x over previous
import functools

import jax
import jax.numpy as jnp
from jax import lax
from jax.experimental import pallas as pl
from jax.experimental.pallas import tpu as pltpu
from jax.experimental.pallas import tpu_sc as plsc

VOCAB = 100000
DESC = 768
PROJ = 128
BATCH = 16384


@functools.cache
def _make_sc_gather(B, D):
    NC, NS = 2, 16
    NW = NC * NS
    b_per_w = B // NW
    C = 64
    n_chunks = b_per_w // C
    mesh = plsc.VectorSubcoreMesh(core_axis_name="c", subcore_axis_name="s")

    @functools.partial(
        pl.kernel,
        mesh=mesh,
        out_type=jax.ShapeDtypeStruct((B, D), jnp.float32),
        scratch_types=[
            pltpu.VMEM((b_per_w,), jnp.int32),
            pltpu.VMEM((C, D), jnp.float32),
            pltpu.VMEM((C, D), jnp.float32),
            pltpu.SemaphoreType.DMA,
            pltpu.SemaphoreType.DMA,
        ],
    )
    def gather(idx_hbm, table_hbm, out_hbm, idx_v, buf0, buf1, sem0, sem1):
        wid = lax.axis_index("s") * NC + lax.axis_index("c")
        base = wid * b_per_w
        pltpu.sync_copy(idx_hbm.at[pl.ds(base, b_per_w)], idx_v)
        bufs = (buf0, buf1)
        sems = (sem0, sem1)

        pltpu.async_copy(table_hbm.at[idx_v.at[pl.ds(0, C)]], buf0, sem0)

        def chunk(i, carry):
            slot = lax.rem(i, 2)

            def do(b, s, b_next, s_next):
                @pl.when(i + 1 < n_chunks)
                def _():
                    off = pl.multiple_of((i + 1) * C, 8)
                    pltpu.async_copy(
                        table_hbm.at[idx_v.at[pl.ds(off, C)]], b_next, s_next)
                pltpu.make_async_copy(table_hbm.at[idx_v.at[pl.ds(0, C)]], b, s).wait()
                pltpu.sync_copy(b, out_hbm.at[pl.ds(base + i * C, C)])

            @pl.when(slot == 0)
            def _():
                do(buf0, sem0, buf1, sem1)

            @pl.when(slot == 1)
            def _():
                do(buf1, sem1, buf0, sem0)

            return carry

        lax.fori_loop(0, n_chunks, chunk, 0)

    return gather


_BM = 2048


def _proj_body(vals_ref, emb_ref, wproj_ref, wvalt_ref, bval_ref, bproj_ref, out_ref):
    wp1 = wproj_ref[:, :DESC]
    wp2 = wproj_ref[:, DESC:]
    c1 = lax.dot_general(wvalt_ref[...], wp1, (((1,), (1,)), ((), ())),
                         preferred_element_type=jnp.float32)
    c0 = lax.dot_general(bval_ref[...], wp1, (((1,), (1,)), ((), ())),
                         preferred_element_type=jnp.float32) + bproj_ref[...]
    emb_term = lax.dot_general(emb_ref[...], wp2, (((1,), (1,)), ((), ())),
                               preferred_element_type=jnp.float32)
    val_term = lax.dot_general(vals_ref[...], c1, (((1,), (0,)), ((), ())),
                               preferred_element_type=jnp.float32)
    out_ref[...] = emb_term + val_term + c0


def _proj(values, emb, W_proj, W_val_t, b_val2, b_proj2):
    grid = (BATCH // _BM,)
    return pl.pallas_call(
        _proj_body,
        grid=grid,
        in_specs=[
            pl.BlockSpec((_BM, 1), lambda i: (i, 0)),
            pl.BlockSpec((_BM, DESC), lambda i: (i, 0)),
            pl.BlockSpec((PROJ, 2 * DESC), lambda i: (0, 0)),
            pl.BlockSpec((1, DESC), lambda i: (0, 0)),
            pl.BlockSpec((1, DESC), lambda i: (0, 0)),
            pl.BlockSpec((1, PROJ), lambda i: (0, 0)),
        ],
        out_specs=pl.BlockSpec((_BM, PROJ), lambda i: (i, 0)),
        out_shape=jax.ShapeDtypeStruct((BATCH, PROJ), jnp.float32),
    )(values, emb, W_proj, W_val_t, b_val2, b_proj2)


def kernel(description_idx, values, embedded_descriptions, W_val, b_val, W_proj, b_proj):
    idx = description_idx.astype(jnp.int32)
    emb = _make_sc_gather(BATCH, DESC)(idx, embedded_descriptions)
    return _proj(
        values,
        emb,
        W_proj,
        W_val.reshape(1, DESC),
        b_val.reshape(1, DESC),
        b_proj.reshape(1, PROJ),
    )
